# baseline (speedup 1.0000x reference)
import functools
import jax
import jax.numpy as jnp
from jax import lax
from jax.experimental import pallas as pl
from jax.experimental.pallas import tpu as pltpu
from jax.experimental.pallas import tpu_sc as plsc

_K = 40
_NBUF = 4


@functools.partial(jax.jit, static_argnames=("n", "m", "d"))
def _pool(x, idx_flat, n, m, d):
    info = plsc.get_sparse_core_info()
    nc, ns, lanes = info.num_cores, info.num_subcores, info.num_lanes
    nw = nc * ns
    k = _K
    nbuf = _NBUF
    t_total = m // k
    n_max = -(-t_total // nw)
    n_grp = -(-n_max // nbuf)
    vecs = d // lanes

    mesh = plsc.VectorSubcoreMesh(core_axis_name="c", subcore_axis_name="s")

    @functools.partial(
        pl.kernel,
        out_type=jax.ShapeDtypeStruct((n + m, d), jnp.float32),
        mesh=mesh,
        scratch_types=(
            [pltpu.VMEM((2 * k, d), jnp.float32) for _ in range(nbuf)]
            + [pltpu.VMEM((n_max * 2 * k,), jnp.int32)]
            + [pltpu.SemaphoreType.DMA for _ in range(2 * nbuf)]
        ),
    )
    def sc_kernel(x_hbm, i_hbm, out_hbm, *scr):
        buf = scr[:nbuf]
        i_v = scr[nbuf]
        sem_g = scr[nbuf + 1:nbuf + 1 + nbuf]
        sem_s = scr[nbuf + 1 + nbuf:]

        wid = lax.axis_index("s") * nc + lax.axis_index("c")
        t0 = wid * t_total // nw
        t1 = (wid + 1) * t_total // nw
        n_loc = t1 - t0

        def wait_store(b):
            pltpu.make_async_copy(buf[b].at[pl.ds(0, k)],
                                  out_hbm.at[pl.ds(0, k)], sem_s[b]).wait()

        def copy_grp(g, carry):
            for b in range(nbuf):
                j = g * nbuf + b
                t = t0 + j

                @pl.when(jnp.logical_and(g > 0, (g - 1) * nbuf + b < n_loc))
                def _():
                    wait_store(b)

                @pl.when(j < n_loc)
                def _():
                    pltpu.async_copy(x_hbm.at[pl.ds(t * k, k)],
                                     buf[b].at[pl.ds(0, k)], sem_g[b])
            for b in range(nbuf):
                j = g * nbuf + b
                t = t0 + j

                @pl.when(j < n_loc)
                def _():
                    pltpu.make_async_copy(x_hbm.at[pl.ds(0, k)],
                                          buf[b].at[pl.ds(0, k)],
                                          sem_g[b]).wait()
                    pltpu.async_copy(buf[b].at[pl.ds(0, k)],
                                     out_hbm.at[pl.ds(t * k, k)], sem_s[b])
            return carry

        lax.fori_loop(0, n_grp, copy_grp, 0)
        for b in range(nbuf):
            @pl.when((n_grp - 1) * nbuf + b < n_loc)
            def _():
                wait_store(b)

        pltpu.sync_copy(i_hbm.at[pl.ds(t0 * 2 * k, n_max * 2 * k)], i_v)

        def avg_inplace(bb):
            def row_body(r, c):
                for j in range(vecs):
                    sl = pl.ds(j * lanes, lanes)
                    bb[r, sl] = (bb[2 * r, sl] + bb[2 * r + 1, sl]) * 0.5
                return c
            lax.fori_loop(0, k, row_body, 0)

        def pool_grp(g, carry):
            for b in range(nbuf):
                j = g * nbuf + b

                @pl.when(jnp.logical_and(g > 0, (g - 1) * nbuf + b < n_loc))
                def _():
                    wait_store(b)

                @pl.when(j < n_loc)
                def _():
                    pltpu.async_copy(x_hbm.at[i_v.at[pl.ds(j * 2 * k, 2 * k)]],
                                     buf[b], sem_g[b])
            for b in range(nbuf):
                j = g * nbuf + b
                t = t0 + j

                @pl.when(j < n_loc)
                def _():
                    pltpu.make_async_copy(x_hbm.at[i_v.at[pl.ds(0, 2 * k)]],
                                          buf[b], sem_g[b]).wait()
                    avg_inplace(buf[b])
                    pltpu.async_copy(buf[b].at[pl.ds(0, k)],
                                     out_hbm.at[pl.ds(n + t * k, k)],
                                     sem_s[b])
            return carry

        lax.fori_loop(0, n_grp, pool_grp, 0)
        for b in range(nbuf):
            @pl.when((n_grp - 1) * nbuf + b < n_loc)
            def _():
                wait_store(b)

    return sc_kernel(x, idx_flat)


def kernel(X, pool_idx):
    n, d = X.shape
    m = pool_idx.shape[1]
    idx_flat = pool_idx[0].astype(jnp.int32).reshape(-1)
    return _pool(X, idx_flat, n, m, d)


# speedup vs baseline: 2.5019x; 2.5019x over previous
import functools
import jax
import jax.numpy as jnp
from jax import lax
from jax.experimental import pallas as pl
from jax.experimental.pallas import tpu as pltpu
from jax.experimental.pallas import tpu_sc as plsc

_K = 80
_NBUF = 3


@functools.partial(jax.jit, static_argnames=("n", "m", "d"))
def _pool(x, idx0, idx1, n, m, d):
    info = plsc.get_sparse_core_info()
    nc, ns, lanes = info.num_cores, info.num_subcores, info.num_lanes
    nw = nc * ns
    k = _K
    nbuf = _NBUF
    t_total = m // k
    n_max = -(-t_total // nw)
    n_grp = -(-n_max // nbuf)
    vecs = d // lanes

    mesh = plsc.VectorSubcoreMesh(core_axis_name="c", subcore_axis_name="s")

    @functools.partial(
        pl.kernel,
        out_type=jax.ShapeDtypeStruct((n + m, d), jnp.float32),
        mesh=mesh,
        scratch_types=(
            [pltpu.VMEM((k, d), jnp.float32) for _ in range(2 * nbuf)]
            + [pltpu.VMEM((n_max * k,), jnp.int32) for _ in range(2)]
            + [pltpu.SemaphoreType.DMA for _ in range(5 * nbuf)]
        ),
    )
    def sc_kernel(x_hbm, i0_hbm, i1_hbm, out_hbm, *scr):
        buf_a = scr[:nbuf]
        buf_b = scr[nbuf:2 * nbuf]
        i0_v, i1_v = scr[2 * nbuf], scr[2 * nbuf + 1]
        sems = scr[2 * nbuf + 2:]
        sem_a = sems[:nbuf]
        sem_b = sems[nbuf:2 * nbuf]
        sem_ps = sems[2 * nbuf:3 * nbuf]
        sem_cl = sems[3 * nbuf:4 * nbuf]
        sem_cs = sems[4 * nbuf:]

        wid = lax.axis_index("s") * nc + lax.axis_index("c")
        t0 = wid * t_total // nw
        t1 = (wid + 1) * t_total // nw
        n_loc = t1 - t0

        def avg_inplace(ba, bb):
            def row_body(r, c):
                for j in range(vecs):
                    sl = pl.ds(j * lanes, lanes)
                    ba[r, sl] = (ba[r, sl] + bb[r, sl]) * 0.5
                return c
            lax.fori_loop(0, k, row_body, 0)

        def wait_pstore(b):
            pltpu.make_async_copy(buf_a[b], out_hbm.at[pl.ds(0, k)],
                                  sem_ps[b]).wait()

        def wait_cstore(b):
            pltpu.make_async_copy(buf_b[b], out_hbm.at[pl.ds(0, k)],
                                  sem_cs[b]).wait()

        pltpu.sync_copy(i0_hbm.at[pl.ds(t0 * k, n_max * k)], i0_v)
        pltpu.sync_copy(i1_hbm.at[pl.ds(t0 * k, n_max * k)], i1_v)

        def grp(g, carry):
            for b in range(nbuf):
                j = g * nbuf + b

                @pl.when(jnp.logical_and(g > 0, (g - 1) * nbuf + b < n_loc))
                def _():
                    wait_pstore(b)
                    wait_cstore(b)

                @pl.when(j < n_loc)
                def _():
                    pltpu.async_copy(x_hbm.at[i0_v.at[pl.ds(j * k, k)]],
                                     buf_a[b], sem_a[b])
                    pltpu.async_copy(x_hbm.at[i1_v.at[pl.ds(j * k, k)]],
                                     buf_b[b], sem_b[b])
            for b in range(nbuf):
                j = g * nbuf + b
                t = t0 + j

                @pl.when(j < n_loc)
                def _():
                    pltpu.make_async_copy(x_hbm.at[i0_v.at[pl.ds(0, k)]],
                                          buf_a[b], sem_a[b]).wait()
                    pltpu.make_async_copy(x_hbm.at[i1_v.at[pl.ds(0, k)]],
                                          buf_b[b], sem_b[b]).wait()
                    avg_inplace(buf_a[b], buf_b[b])
                    pltpu.async_copy(buf_a[b],
                                     out_hbm.at[pl.ds(n + t * k, k)],
                                     sem_ps[b])
                    pltpu.async_copy(x_hbm.at[pl.ds(t * k, k)], buf_b[b],
                                     sem_cl[b])
            for b in range(nbuf):
                j = g * nbuf + b
                t = t0 + j

                @pl.when(j < n_loc)
                def _():
                    pltpu.make_async_copy(x_hbm.at[pl.ds(0, k)], buf_b[b],
                                          sem_cl[b]).wait()
                    pltpu.async_copy(buf_b[b], out_hbm.at[pl.ds(t * k, k)],
                                     sem_cs[b])
            return carry

        lax.fori_loop(0, n_grp, grp, 0)
        for b in range(nbuf):
            @pl.when((n_grp - 1) * nbuf + b < n_loc)
            def _():
                wait_pstore(b)
                wait_cstore(b)

    return sc_kernel(x, idx0, idx1)


def kernel(X, pool_idx):
    n, d = X.shape
    m = pool_idx.shape[1]
    idx = pool_idx[0].astype(jnp.int32)
    return _pool(X, idx[:, 0], idx[:, 1], n, m, d)


# speedup vs baseline: 2.5122x; 1.0041x over previous
import functools
import jax
import jax.numpy as jnp
from jax import lax
from jax.experimental import pallas as pl
from jax.experimental.pallas import tpu as pltpu
from jax.experimental.pallas import tpu_sc as plsc

_K = 80
_NBUF = 2


@functools.partial(jax.jit, static_argnames=("n", "m", "d"))
def _pool(x, idx0, idx1, n, m, d):
    info = plsc.get_sparse_core_info()
    nc, ns, lanes = info.num_cores, info.num_subcores, info.num_lanes
    nw = nc * ns
    k = _K
    nbuf = _NBUF
    t_total = m // k
    n_max = -(-t_total // nw)
    n_grp = -(-n_max // nbuf)
    vecs = d // lanes

    mesh = plsc.VectorSubcoreMesh(core_axis_name="c", subcore_axis_name="s")

    @functools.partial(
        pl.kernel,
        out_type=jax.ShapeDtypeStruct((n + m, d), jnp.float32),
        mesh=mesh,
        scratch_types=(
            [pltpu.VMEM((k, d), jnp.float32) for _ in range(3 * nbuf)]
            + [pltpu.VMEM((n_max * k,), jnp.int32) for _ in range(2)]
            + [pltpu.SemaphoreType.DMA for _ in range(5 * nbuf)]
        ),
    )
    def sc_kernel(x_hbm, i0_hbm, i1_hbm, out_hbm, *scr):
        buf_a = scr[:nbuf]
        buf_b = scr[nbuf:2 * nbuf]
        buf_c = scr[2 * nbuf:3 * nbuf]
        i0_v, i1_v = scr[3 * nbuf], scr[3 * nbuf + 1]
        sems = scr[3 * nbuf + 2:]
        sem_a = sems[:nbuf]
        sem_b = sems[nbuf:2 * nbuf]
        sem_ps = sems[2 * nbuf:3 * nbuf]
        sem_cl = sems[3 * nbuf:4 * nbuf]
        sem_cs = sems[4 * nbuf:]

        wid = lax.axis_index("s") * nc + lax.axis_index("c")
        t0 = wid * t_total // nw
        t1 = (wid + 1) * t_total // nw
        n_loc = t1 - t0

        def avg_inplace(ba, bb):
            def row_body(r, c):
                for j in range(vecs):
                    sl = pl.ds(j * lanes, lanes)
                    ba[r, sl] = (ba[r, sl] + bb[r, sl]) * 0.5
                return c
            lax.fori_loop(0, k, row_body, 0)

        def wait_pstore(b):
            pltpu.make_async_copy(buf_a[b], out_hbm.at[pl.ds(0, k)],
                                  sem_ps[b]).wait()

        def wait_cstore(b):
            pltpu.make_async_copy(buf_c[b], out_hbm.at[pl.ds(0, k)],
                                  sem_cs[b]).wait()

        pltpu.sync_copy(i0_hbm.at[pl.ds(t0 * k, n_max * k)], i0_v)
        pltpu.sync_copy(i1_hbm.at[pl.ds(t0 * k, n_max * k)], i1_v)

        def grp(g, carry):
            for b in range(nbuf):
                j = g * nbuf + b
                t = t0 + j

                @pl.when(jnp.logical_and(g > 0, (g - 1) * nbuf + b < n_loc))
                def _():
                    wait_pstore(b)
                    wait_cstore(b)

                @pl.when(j < n_loc)
                def _():
                    pltpu.async_copy(x_hbm.at[i0_v.at[pl.ds(j * k, k)]],
                                     buf_a[b], sem_a[b])
                    pltpu.async_copy(x_hbm.at[i1_v.at[pl.ds(j * k, k)]],
                                     buf_b[b], sem_b[b])
                    pltpu.async_copy(x_hbm.at[pl.ds(t * k, k)], buf_c[b],
                                     sem_cl[b])
            for b in range(nbuf):
                j = g * nbuf + b
                t = t0 + j

                @pl.when(j < n_loc)
                def _():
                    pltpu.make_async_copy(x_hbm.at[pl.ds(0, k)], buf_c[b],
                                          sem_cl[b]).wait()
                    pltpu.async_copy(buf_c[b], out_hbm.at[pl.ds(t * k, k)],
                                     sem_cs[b])
                    pltpu.make_async_copy(x_hbm.at[i0_v.at[pl.ds(0, k)]],
                                          buf_a[b], sem_a[b]).wait()
                    pltpu.make_async_copy(x_hbm.at[i1_v.at[pl.ds(0, k)]],
                                          buf_b[b], sem_b[b]).wait()
                    avg_inplace(buf_a[b], buf_b[b])
                    pltpu.async_copy(buf_a[b],
                                     out_hbm.at[pl.ds(n + t * k, k)],
                                     sem_ps[b])
            return carry

        lax.fori_loop(0, n_grp, grp, 0)
        for b in range(nbuf):
            @pl.when((n_grp - 1) * nbuf + b < n_loc)
            def _():
                wait_pstore(b)
                wait_cstore(b)

    return sc_kernel(x, idx0, idx1)


def kernel(X, pool_idx):
    n, d = X.shape
    m = pool_idx.shape[1]
    idx = pool_idx[0].astype(jnp.int32)
    return _pool(X, idx[:, 0], idx[:, 1], n, m, d)
